# baseline (speedup 1.0000x reference)
import jax
import jax.numpy as jnp
from jax.experimental import pallas as pl
from jax.experimental.pallas import tpu as pltpu

_BB = 512


def _half(x, w1_ref, ss_ref, w2_ref, b2_ref, w3_ref, b3_ref, w4_ref, b4_ref):
    f32 = jnp.float32
    bf16 = jnp.bfloat16
    bt = x.shape[0]

    xp_ = jnp.pad(x, ((0, 0), (0, 0), (0, 64)),
                  constant_values=jnp.bfloat16(1.0))
    xc = jnp.concatenate([xp_[:, 0:16], xp_[:, 1:17]], axis=2)
    h1 = jnp.dot(xc.reshape(bt * 16, 512), w1_ref[...],
                 preferred_element_type=f32)
    h1 = jnp.maximum(h1, 0.0).astype(bf16)

    ta = jnp.dot(h1[:, 0:256], ss_ref[...],
                 preferred_element_type=f32).astype(bf16)
    tb = jnp.dot(h1[:, 256:512], ss_ref[...],
                 preferred_element_type=f32).astype(bf16)
    pooled = jnp.maximum(
        jnp.maximum(ta[:, 0:128], ta[:, 128:256]),
        jnp.maximum(tb[:, 0:128], tb[:, 128:256]))

    pm = pooled.reshape(bt, 16, 128)
    pp = jnp.concatenate([pm[:, 1:2], pm, pm[:, 14:15]], axis=1)
    c2 = jnp.concatenate([pp[:, 0:16], pp[:, 1:17], pp[:, 2:18]], axis=2)
    h2 = jnp.dot(c2.reshape(bt * 16, 384), w2_ref[...],
                 preferred_element_type=f32) + b2_ref[...]

    flat = h2.astype(bf16).reshape(bt, 4096)
    y = jnp.dot(flat, w3_ref[...], preferred_element_type=f32) + b3_ref[...]
    y = jnp.maximum(y, 0.0).astype(bf16)
    return jnp.dot(y, w4_ref[...],
                   preferred_element_type=f32) + b4_ref[...]


def _fwd_kernel(x_ref, w1_ref, ss_ref, w2_ref, b2_ref, w3_ref,
                b3_ref, w4_ref, b4_ref, o_ref):
    hb = x_ref.shape[0] // 2
    args = (w1_ref, ss_ref, w2_ref, b2_ref, w3_ref, b3_ref, w4_ref, b4_ref)
    o_ref[0:hb, :] = _half(x_ref[0:hb], *args)
    o_ref[hb:, :] = _half(x_ref[hb:], *args)


def kernel(x, w1, b1, se, so, wpad, w2, b2, w3, b3, w4, b4):
    B = x.shape[0]
    f32 = jnp.float32
    bf16 = jnp.bfloat16

    x32 = jnp.transpose(x, (0, 2, 1, 3)).reshape(B, 32, 96)
    hseq = [1, 0] + list(range(1, 31)) + [31, 30]
    xt = x32[:, jnp.array(hseq), :].reshape(B, 17, 192).astype(bf16)

    bp = ((B + _BB - 1) // _BB) * _BB
    if bp != B:
        xt = jnp.pad(xt, ((0, bp - B), (0, 0), (0, 0)))
    grid = (bp // _BB,)

    w1e = w1[:, 3:99, :]
    w1e = w1e.at[:, 3:6, :].add(w1[:, 0:3, :])
    w1e = w1e.at[:, 90:93, :].add(w1[:, 99:102, :])
    w1cm = w1e.reshape(3, 32, 3, 192).transpose(0, 2, 1, 3).reshape(3, 96, 192)
    W1b = jnp.zeros((512, 512), f32)
    W1b = W1b.at[0:96, 0:192].set(w1cm[0])
    W1b = W1b.at[96:192, 0:192].set(w1cm[1])
    W1b = W1b.at[96:192, 256:448].set(w1cm[0])
    W1b = W1b.at[256:352, 0:192].set(w1cm[2])
    W1b = W1b.at[256:352, 256:448].set(w1cm[1])
    W1b = W1b.at[352:448, 256:448].set(w1cm[2])
    W1b = W1b.at[192, 0:192].set(b1[0])
    W1b = W1b.at[192, 256:448].set(b1[0])
    W1b = W1b.astype(bf16)

    SS4 = jnp.zeros((256, 256), f32)
    SS4 = SS4.at[0:192, 0:96].set(se).at[0:192, 128:224].set(so)
    SS4 = SS4.astype(bf16)

    w2f = jnp.einsum("kj,djn->dkn", wpad, w2)
    w2f = jnp.pad(w2f, ((0, 0), (0, 32), (0, 0))).reshape(384, 256)
    w2f = w2f.astype(bf16)
    w3p = jnp.pad(w3, ((0, 0), (0, 28))).astype(bf16)
    b3p = jnp.pad(b3, ((0, 0), (0, 28)))
    w4p = jnp.pad(w4, ((0, 28), (0, 0))).astype(bf16)

    out = pl.pallas_call(
        _fwd_kernel,
        out_shape=jax.ShapeDtypeStruct((bp, 128), jnp.float32),
        grid=grid,
        in_specs=[
            pl.BlockSpec((_BB, 17, 192), lambda i: (i, 0, 0)),
            pl.BlockSpec((512, 512), lambda i: (0, 0)),
            pl.BlockSpec((256, 256), lambda i: (0, 0)),
            pl.BlockSpec((384, 256), lambda i: (0, 0)),
            pl.BlockSpec((1, 256), lambda i: (0, 0)),
            pl.BlockSpec((4096, 128), lambda i: (0, 0)),
            pl.BlockSpec((1, 128), lambda i: (0, 0)),
            pl.BlockSpec((128, 128), lambda i: (0, 0)),
            pl.BlockSpec((1, 128), lambda i: (0, 0)),
        ],
        out_specs=pl.BlockSpec((_BB, 128), lambda i: (i, 0)),
        compiler_params=pltpu.CompilerParams(
            dimension_semantics=("parallel",),
            vmem_limit_bytes=100 * 1024 * 1024),
    )(xt, W1b, SS4, w2f, b2, w3p, b3p, w4p, b4)
    return out[:B, :10]


# speedup vs baseline: 1.3682x; 1.3682x over previous
import jax
import jax.numpy as jnp
from jax.experimental import pallas as pl
from jax.experimental.pallas import tpu as pltpu

_BB = 512


def _fwd_kernel(x_ref, w1_ref, ss_ref, w2_ref, b2_ref, w3_ref,
                b3_ref, w4_ref, b4_ref, o_ref):
    f32 = jnp.float32
    bf16 = jnp.bfloat16
    bt = x_ref.shape[1]

    x = x_ref[...]
    xp_ = jnp.pad(x, ((0, 0), (0, 0), (0, 64)),
                  constant_values=jnp.bfloat16(1.0))
    xc = jnp.concatenate([xp_[0:16], xp_[1:17]], axis=2)
    h1 = jnp.dot(xc.reshape(16 * bt, 512), w1_ref[...],
                 preferred_element_type=f32)
    h1 = jnp.maximum(h1, 0.0).astype(bf16)

    ta = jnp.dot(h1[:, 0:256], ss_ref[...],
                 preferred_element_type=f32).astype(bf16)
    tb = jnp.dot(h1[:, 256:512], ss_ref[...],
                 preferred_element_type=f32).astype(bf16)
    pooled = jnp.maximum(
        jnp.maximum(ta[:, 0:128], ta[:, 128:256]),
        jnp.maximum(tb[:, 0:128], tb[:, 128:256]))

    pmh = pooled.reshape(16, bt, 128)
    pph = jnp.concatenate([pmh[1:2], pmh, pmh[14:15]], axis=0)
    c2 = jnp.concatenate([pph[0:16], pph[1:17], pph[2:18]], axis=2)
    h2 = jnp.dot(c2.reshape(16 * bt, 384), w2_ref[...],
                 preferred_element_type=f32) + b2_ref[...]

    h2v = h2.astype(bf16).reshape(16, bt, 256)
    acc = jnp.dot(h2v[0], w3_ref[0], preferred_element_type=f32)
    for h in range(1, 16):
        acc = acc + jnp.dot(h2v[h], w3_ref[h], preferred_element_type=f32)
    y = jnp.maximum(acc + b3_ref[...], 0.0).astype(bf16)
    o_ref[...] = jnp.dot(y, w4_ref[...],
                         preferred_element_type=f32) + b4_ref[...]


def kernel(x, w1, b1, se, so, wpad, w2, b2, w3, b3, w4, b4):
    B = x.shape[0]
    f32 = jnp.float32
    bf16 = jnp.bfloat16

    x32 = jnp.transpose(x, (0, 2, 1, 3)).reshape(B, 32, 96)
    hseq = [1, 0] + list(range(1, 31)) + [31, 30]
    xt = x32[:, jnp.array(hseq), :].reshape(B, 17, 192).astype(bf16)
    xt = xt.transpose(1, 0, 2)

    bp = ((B + _BB - 1) // _BB) * _BB
    if bp != B:
        xt = jnp.pad(xt, ((0, 0), (0, bp - B), (0, 0)))
    grid = (bp // _BB,)

    w1e = w1[:, 3:99, :]
    w1e = w1e.at[:, 3:6, :].add(w1[:, 0:3, :])
    w1e = w1e.at[:, 90:93, :].add(w1[:, 99:102, :])
    w1cm = w1e.reshape(3, 32, 3, 192).transpose(0, 2, 1, 3).reshape(3, 96, 192)
    W1b = jnp.zeros((512, 512), f32)
    W1b = W1b.at[0:96, 0:192].set(w1cm[0])
    W1b = W1b.at[96:192, 0:192].set(w1cm[1])
    W1b = W1b.at[96:192, 256:448].set(w1cm[0])
    W1b = W1b.at[256:352, 0:192].set(w1cm[2])
    W1b = W1b.at[256:352, 256:448].set(w1cm[1])
    W1b = W1b.at[352:448, 256:448].set(w1cm[2])
    W1b = W1b.at[192, 0:192].set(b1[0])
    W1b = W1b.at[192, 256:448].set(b1[0])
    W1b = W1b.astype(bf16)

    SS4 = jnp.zeros((256, 256), f32)
    SS4 = SS4.at[0:192, 0:96].set(se).at[0:192, 128:224].set(so)
    SS4 = SS4.astype(bf16)

    w2f = jnp.einsum("kj,djn->dkn", wpad, w2)
    w2f = jnp.pad(w2f, ((0, 0), (0, 32), (0, 0))).reshape(384, 256)
    w2f = w2f.astype(bf16)
    w3p = jnp.pad(w3, ((0, 0), (0, 28))).astype(bf16).reshape(16, 256, 128)
    b3p = jnp.pad(b3, ((0, 0), (0, 28)))
    w4p = jnp.pad(w4, ((0, 28), (0, 0))).astype(bf16)

    out = pl.pallas_call(
        _fwd_kernel,
        out_shape=jax.ShapeDtypeStruct((bp, 128), jnp.float32),
        grid=grid,
        in_specs=[
            pl.BlockSpec((17, _BB, 192), lambda i: (0, i, 0)),
            pl.BlockSpec((512, 512), lambda i: (0, 0)),
            pl.BlockSpec((256, 256), lambda i: (0, 0)),
            pl.BlockSpec((384, 256), lambda i: (0, 0)),
            pl.BlockSpec((1, 256), lambda i: (0, 0)),
            pl.BlockSpec((16, 256, 128), lambda i: (0, 0, 0)),
            pl.BlockSpec((1, 128), lambda i: (0, 0)),
            pl.BlockSpec((128, 128), lambda i: (0, 0)),
            pl.BlockSpec((1, 128), lambda i: (0, 0)),
        ],
        out_specs=pl.BlockSpec((_BB, 128), lambda i: (i, 0)),
        compiler_params=pltpu.CompilerParams(
            dimension_semantics=("parallel",),
            vmem_limit_bytes=100 * 1024 * 1024),
    )(xt, W1b, SS4, w2f, b2, w3p, b3p, w4p, b4)
    return out[:B, :10]
